# baseline (speedup 1.0000x reference)
import functools

import jax
import jax.numpy as jnp
from jax import lax
from jax.experimental import pallas as pl
from jax.experimental.pallas import tpu as pltpu
from jax.experimental.pallas import tpu_sc as plsc

B = 8
C = 128
N = 1024
KS = (8, 16, 32)
PER = 64
OUT_C = 192
BN = B * N

SC_CORES = 2
SC_SUBCORES = 16
NW = SC_CORES * SC_SUBCORES
PTS_PER_W = BN // NW
GROUP_ROWS = 128

_HIGH = jax.lax.Precision.HIGHEST


def _stage1_body(x_ref, xt_ref, w_ref, idx_ref, u_ref, v_ref, d_ref):
    b = pl.program_id(0)
    x = x_ref[0]
    xt = xt_ref[0]

    ah = xt.astype(jnp.bfloat16)
    bh = x.astype(jnp.bfloat16)
    inner = -2.0 * lax.dot_general(ah, bh, (((1,), (0,)), ((), ())),
                                   preferred_element_type=jnp.float32)
    xx_row = jnp.sum(x * x, axis=0, keepdims=True)
    xx_col = jnp.sum(xt * xt, axis=1, keepdims=True)
    d_ref[...] = (-xx_row - inner) - xx_col

    cols = lax.broadcasted_iota(jnp.int32, (N, N), 1)
    jj = lax.broadcasted_iota(jnp.int32, (N, KS[-1]), 1)
    base = b * N

    rm0 = jnp.max(d_ref[...], axis=1, keepdims=True)

    def body(j, carry):
        idxc, rm = carry
        d = d_ref[...]
        am = jnp.min(jnp.where(d == rm, cols, N), axis=1, keepdims=True)
        dn = jnp.where(cols == am, jnp.float32(-1e30), d)
        d_ref[...] = dn
        rm_next = jnp.max(dn, axis=1, keepdims=True)
        return jnp.where(jj == j, am + base, idxc), rm_next

    idx_ref[0], _ = lax.fori_loop(
        0, KS[-1], body, (jnp.zeros((N, KS[-1]), jnp.int32), rm0))

    for i in range(3):
        w = w_ref[i]
        a = w[:, :C]
        bm = w[:, C:] - a
        u_ref[0, i] = lax.dot_general(xt, a, (((1,), (1,)), ((), ())),
                                      preferred_element_type=jnp.float32,
                                      precision=_HIGH)
        v_ref[0, i] = lax.dot_general(xt, bm, (((1,), (1,)), ((), ())),
                                      preferred_element_type=jnp.float32,
                                      precision=_HIGH)


def _stage1(x, xt, w_all):
    return pl.pallas_call(
        _stage1_body,
        grid=(B,),
        in_specs=[
            pl.BlockSpec((1, C, N), lambda b: (b, 0, 0)),
            pl.BlockSpec((1, N, C), lambda b: (b, 0, 0)),
            pl.BlockSpec((3, PER, 2 * C), lambda b: (0, 0, 0)),
        ],
        out_specs=[
            pl.BlockSpec((1, N, KS[-1]), lambda b: (b, 0, 0)),
            pl.BlockSpec((1, 3, N, PER), lambda b: (b, 0, 0, 0)),
            pl.BlockSpec((1, 3, N, PER), lambda b: (b, 0, 0, 0)),
        ],
        out_shape=[
            jax.ShapeDtypeStruct((B, N, KS[-1]), jnp.int32),
            jax.ShapeDtypeStruct((B, 3, N, PER), jnp.float32),
            jax.ShapeDtypeStruct((B, 3, N, PER), jnp.float32),
        ],
        scratch_shapes=[pltpu.VMEM((N, N), jnp.float32)],
    )(x, xt, w_all)


def _sc_body(i0, i1, i2, t0, t1, t2,
             o_mx0, o_mn0, o_s10, o_s20,
             o_mx1, o_mn1, o_s11, o_s21,
             o_mx2, o_mn2, o_s12, o_s22,
             idx_v, rows_v, mx_v, mn_v, s1_v, s2_v, sem):
    wid = lax.axis_index("s") * SC_CORES + lax.axis_index("c")
    base_pt = wid * PTS_PER_W

    scales = [
        (KS[0], i0, t0, o_mx0, o_mn0, o_s10, o_s20),
        (KS[1], i1, t1, o_mx1, o_mn1, o_s11, o_s21),
        (KS[2], i2, t2, o_mx2, o_mn2, o_s12, o_s22),
    ]
    for k, iflat, tab, o_mx, o_mn, o_s1, o_s2 in scales:
        gpts = GROUP_ROWS // k
        ngroups = PTS_PER_W // gpts

        def group(g, _, k=k, iflat=iflat, tab=tab, gpts=gpts):
            p0 = base_pt + g * gpts
            pltpu.sync_copy(iflat.at[pl.ds(p0 * k, GROUP_ROWS)], idx_v)
            pltpu.async_copy(tab.at[idx_v], rows_v, sem).wait()
            for p in range(gpts):
                r0 = p * k
                acc = []
                for c in range(4):
                    val = rows_v[r0, pl.ds(c * 16, 16)]
                    acc += [val, val, val, val * val]

                def red(j, a, r0=r0):
                    out = []
                    for c in range(4):
                        val = rows_v[r0 + j, pl.ds(c * 16, 16)]
                        m, mn, s1, s2 = a[4 * c:4 * c + 4]
                        out += [jnp.maximum(m, val), jnp.minimum(mn, val),
                                s1 + val, s2 + val * val]
                    return tuple(out)

                acc = lax.fori_loop(1, k, red, tuple(acc))
                lp = g * gpts + p
                for c in range(4):
                    sl = pl.ds(c * 16, 16)
                    mx_v[lp, sl] = acc[4 * c]
                    mn_v[lp, sl] = acc[4 * c + 1]
                    s1_v[lp, sl] = acc[4 * c + 2]
                    s2_v[lp, sl] = acc[4 * c + 3]
            return 0

        lax.fori_loop(0, ngroups, group, 0)
        rows = pl.ds(base_pt, PTS_PER_W)
        pltpu.sync_copy(mx_v, o_mx.at[rows])
        pltpu.sync_copy(mn_v, o_mn.at[rows])
        pltpu.sync_copy(s1_v, o_s1.at[rows])
        pltpu.sync_copy(s2_v, o_s2.at[rows])


def _stage2(i0, i1, i2, t0, t1, t2):
    mesh = plsc.VectorSubcoreMesh(core_axis_name="c", subcore_axis_name="s",
                                  num_cores=SC_CORES, num_subcores=SC_SUBCORES)
    fn = pl.kernel(
        _sc_body,
        out_type=[jax.ShapeDtypeStruct((BN, PER), jnp.float32)] * 12,
        mesh=mesh,
        scratch_types=[
            pltpu.VMEM((GROUP_ROWS,), jnp.int32),
            pltpu.VMEM((GROUP_ROWS, PER), jnp.float32),
            pltpu.VMEM((PTS_PER_W, PER), jnp.float32),
            pltpu.VMEM((PTS_PER_W, PER), jnp.float32),
            pltpu.VMEM((PTS_PER_W, PER), jnp.float32),
            pltpu.VMEM((PTS_PER_W, PER), jnp.float32),
            pltpu.SemaphoreType.DMA,
        ],
        compiler_params=pltpu.CompilerParams(use_tc_tiling_on_sc=False),
    )
    return fn(i0, i1, i2, t0, t1, t2)


_RBLK = 1024
_NBLK = BN // _RBLK


def _rows_spec(width):
    return pl.BlockSpec((_RBLK, width), lambda i: (i, 0))


def _fixed_spec(shape):
    nd = len(shape)
    return pl.BlockSpec(shape, lambda i: (0,) * nd)


def _s3a_body(s10, s20, v0, s11, s21, v1, s12, s22, v2, out_ref):
    rows = []
    for s1, s2, v in ((s10, s20, v0), (s11, s21, v1), (s12, s22, v2)):
        s1_v = s1[...]
        s2_v = s2[...]
        vv = v[...]
        rows += [jnp.sum(s1_v, axis=0, keepdims=True),
                 jnp.sum(s2_v, axis=0, keepdims=True),
                 jnp.sum(vv * s1_v, axis=0, keepdims=True),
                 jnp.sum(vv, axis=0, keepdims=True),
                 jnp.sum(vv * vv, axis=0, keepdims=True)]
    blk = jnp.concatenate(rows, axis=0)

    @pl.when(pl.program_id(0) == 0)
    def _():
        out_ref[...] = jnp.zeros_like(out_ref)

    out_ref[...] += blk


def _s3b_body(stats, gam, bet, mx0, mn0, v0, mx1, mn1, v1, mx2, mn2, v2,
              h_ref):
    parts = []
    groups = ((mx0, mn0, v0), (mx1, mn1, v1), (mx2, mn2, v2))
    for i, (mx, mn, v) in enumerate(groups):
        k = KS[i]
        cnt = jnp.float32(BN * k)
        S1 = stats[5 * i:5 * i + 1, :]
        S2 = stats[5 * i + 1:5 * i + 2, :]
        X = stats[5 * i + 2:5 * i + 3, :]
        V1 = stats[5 * i + 3:5 * i + 4, :]
        V2 = stats[5 * i + 4:5 * i + 5, :]
        mean = (S1 + k * V1) / cnt
        e2 = (S2 + 2.0 * X + k * V2) / cnt
        var = e2 - mean * mean
        gamma = gam[i:i + 1, :]
        beta = bet[i:i + 1, :]
        pre = jnp.where(gamma >= 0.0, mx[...], mn[...]) + v[...]
        y = (pre - mean) * (gamma / jnp.sqrt(var + 1e-5)) + beta
        parts.append(jnp.where(y > 0.0, y, 0.2 * y))
    h_ref[...] = jnp.concatenate(parts, axis=1)


def _s3c_body(h_ref, wf_ref, bf_ref, z_ref, acc_ref):
    z = lax.dot_general(h_ref[...], wf_ref[...], (((1,), (1,)), ((), ())),
                        preferred_element_type=jnp.float32, precision=_HIGH)
    z = z + bf_ref[...]
    z_ref[...] = z

    @pl.when(pl.program_id(0) == 0)
    def _():
        acc_ref[...] = jnp.zeros_like(acc_ref)

    acc_ref[...] += jnp.concatenate(
        [jnp.sum(z, axis=0, keepdims=True),
         jnp.sum(z * z, axis=0, keepdims=True)], axis=0)


def _s3d_body(z_ref, acc_ref, gf_ref, betaf_ref, out_ref):
    mean = acc_ref[0:1, :] / jnp.float32(BN)
    var = acc_ref[1:2, :] / jnp.float32(BN) - mean * mean
    z = (z_ref[...] - mean) * (gf_ref[...] / jnp.sqrt(var + 1e-5))
    z = z + betaf_ref[...]
    out_ref[...] = 0.5 * z * (1.0 + lax.erf(z * jnp.float32(0.7071067811865476)))


def _stage3(sc_outs, v_rows, gam, bet, wf, bf, gf, betaf):
    (mx0, mn0, s10, s20, mx1, mn1, s11, s21, mx2, mn2, s12, s22) = sc_outs
    v0, v1, v2 = v_rows

    stats = pl.pallas_call(
        _s3a_body,
        grid=(_NBLK,),
        in_specs=[_rows_spec(PER)] * 9,
        out_specs=_fixed_spec((15, PER)),
        out_shape=jax.ShapeDtypeStruct((15, PER), jnp.float32),
    )(s10, s20, v0, s11, s21, v1, s12, s22, v2)

    h = pl.pallas_call(
        _s3b_body,
        grid=(_NBLK,),
        in_specs=[_fixed_spec((15, PER)), _fixed_spec((3, PER)),
                  _fixed_spec((3, PER))] + [_rows_spec(PER)] * 9,
        out_specs=_rows_spec(OUT_C),
        out_shape=jax.ShapeDtypeStruct((BN, OUT_C), jnp.float32),
    )(stats, gam, bet, mx0, mn0, v0, mx1, mn1, v1, mx2, mn2, v2)

    z, acc = pl.pallas_call(
        _s3c_body,
        grid=(_NBLK,),
        in_specs=[_rows_spec(OUT_C), _fixed_spec((OUT_C, OUT_C)),
                  _fixed_spec((1, OUT_C))],
        out_specs=[_rows_spec(OUT_C), _fixed_spec((2, OUT_C))],
        out_shape=[jax.ShapeDtypeStruct((BN, OUT_C), jnp.float32),
                   jax.ShapeDtypeStruct((2, OUT_C), jnp.float32)],
    )(h, wf, bf)

    return pl.pallas_call(
        _s3d_body,
        grid=(_NBLK,),
        in_specs=[_rows_spec(OUT_C), _fixed_spec((2, OUT_C)),
                  _fixed_spec((1, OUT_C)), _fixed_spec((1, OUT_C))],
        out_specs=_rows_spec(OUT_C),
        out_shape=jax.ShapeDtypeStruct((BN, OUT_C), jnp.float32),
    )(z, acc, gf, betaf)


def kernel(x, W0, gamma0, beta0, W1, gamma1, beta1, W2, gamma2, beta2,
           Wf, bf, gf, betaf):
    xt = jnp.transpose(x, (0, 2, 1))
    w_all = jnp.stack([W0, W1, W2])
    idxg, u, v = _stage1(x, xt, w_all)

    i2 = idxg.reshape(-1)
    i1 = idxg[:, :, :KS[1]].reshape(-1)
    i0 = idxg[:, :, :KS[0]].reshape(-1)
    t0 = u[:, 0].reshape(BN, PER)
    t1 = u[:, 1].reshape(BN, PER)
    t2 = u[:, 2].reshape(BN, PER)
    sc_outs = _stage2(i0, i1, i2, t0, t1, t2)

    v_rows = [v[:, i].reshape(BN, PER) for i in range(3)]
    gam = jnp.stack([gamma0, gamma1, gamma2])
    bet = jnp.stack([beta0, beta1, beta2])
    y = _stage3(sc_outs, v_rows, gam, bet, Wf,
                bf.reshape(1, OUT_C), gf.reshape(1, OUT_C),
                betaf.reshape(1, OUT_C))
    return jnp.transpose(y.reshape(B, N, OUT_C), (0, 2, 1))


# speedup vs baseline: 1.1730x; 1.1730x over previous
import jax
import jax.numpy as jnp
from jax import lax
from jax.experimental import pallas as pl
from jax.experimental.pallas import tpu as pltpu
from jax.experimental.pallas import tpu_sc as plsc

B = 8
C = 128
N = 1024
KS = (8, 16, 32)
PER = 64
OUT_C = 192
BN = B * N
CHUNKS = 4
CB = B // CHUNKS
CPTS = CB * N

SC_CORES = 2
SC_SUBCORES = 16
NW = SC_CORES * SC_SUBCORES
WPTS = CPTS // NW
GROUP_ROWS = 128

_HIGH = jax.lax.Precision.HIGHEST


def _uv_body(xt_ref, w_ref, u_ref, v_ref, st_ref):
    xt = xt_ref[0]

    @pl.when(pl.program_id(0) == 0)
    def _():
        st_ref[...] = jnp.zeros_like(st_ref)

    acc = []
    for i in range(3):
        w = w_ref[i]
        a = w[:, :C]
        bm = w[:, C:] - a
        u_ref[0, i] = lax.dot_general(xt, a, (((1,), (1,)), ((), ())),
                                      preferred_element_type=jnp.float32,
                                      precision=_HIGH)
        vv = lax.dot_general(xt, bm, (((1,), (1,)), ((), ())),
                             preferred_element_type=jnp.float32,
                             precision=_HIGH)
        v_ref[0, i] = vv
        acc.append(jnp.sum(vv, axis=0, keepdims=True))
        acc.append(jnp.sum(vv * vv, axis=0, keepdims=True))
    st_ref[...] += jnp.concatenate(acc, axis=0)


def _uv(xt, w_all):
    return pl.pallas_call(
        _uv_body,
        grid=(B,),
        in_specs=[
            pl.BlockSpec((1, N, C), lambda b: (b, 0, 0)),
            pl.BlockSpec((3, PER, 2 * C), lambda b: (0, 0, 0)),
        ],
        out_specs=[
            pl.BlockSpec((1, 3, N, PER), lambda b: (b, 0, 0, 0)),
            pl.BlockSpec((1, 3, N, PER), lambda b: (b, 0, 0, 0)),
            pl.BlockSpec((6, PER), lambda b: (0, 0)),
        ],
        out_shape=[
            jax.ShapeDtypeStruct((B, 3, N, PER), jnp.float32),
            jax.ShapeDtypeStruct((B, 3, N, PER), jnp.float32),
            jax.ShapeDtypeStruct((6, PER), jnp.float32),
        ],
    )(xt, w_all)


def _make_top_body(chunk):
    def body(x_ref, xt_ref, idx_ref, d_ref):
        b = pl.program_id(0)
        x = x_ref[0]
        xt = xt_ref[0]

        ah = xt.astype(jnp.bfloat16)
        bh = x.astype(jnp.bfloat16)
        inner = -2.0 * lax.dot_general(ah, bh, (((1,), (0,)), ((), ())),
                                       preferred_element_type=jnp.float32)
        xx_row = jnp.sum(x * x, axis=0, keepdims=True)
        xx_col = jnp.sum(xt * xt, axis=1, keepdims=True)
        d_ref[...] = (-xx_row - inner) - xx_col

        cols = lax.broadcasted_iota(jnp.int32, (N, N), 1)
        jj = lax.broadcasted_iota(jnp.int32, (N, KS[-1]), 1)
        base = (chunk * CB + b) * N

        def step(j, idxc):
            d = d_ref[...]
            rm = jnp.max(d, axis=1, keepdims=True)
            am = jnp.min(jnp.where(d == rm, cols, N), axis=1, keepdims=True)
            d_ref[...] = jnp.where(cols == am, jnp.float32(-1e30), d)
            return jnp.where(jj == j, am + base, idxc)

        idx_ref[0] = lax.fori_loop(0, KS[-1], step,
                                   jnp.zeros((N, KS[-1]), jnp.int32))
    return body


def _topk(x_c, xt_c, chunk):
    return pl.pallas_call(
        _make_top_body(chunk),
        grid=(CB,),
        in_specs=[
            pl.BlockSpec((1, C, N), lambda b: (b, 0, 0)),
            pl.BlockSpec((1, N, C), lambda b: (b, 0, 0)),
        ],
        out_specs=pl.BlockSpec((1, N, KS[-1]), lambda b: (b, 0, 0)),
        out_shape=jax.ShapeDtypeStruct((CB, N, KS[-1]), jnp.int32),
        scratch_shapes=[pltpu.VMEM((N, N), jnp.float32)],
    )(x_c, xt_c)


def _sc_body(i0, i1, i2, t0, t1, t2, v0c, v1c, v2c,
             o_mx0, o_mn0, o_mx1, o_mn1, o_mx2, o_mn2, o_part,
             idx_a, idx_b, rows_a, rows_b, v_v, mx_v, mn_v, part_v,
             sem_a, sem_b):
    wid = lax.axis_index("s") * SC_CORES + lax.axis_index("c")
    base_pt = wid * WPTS

    scales = [
        (0, KS[0], i0, t0, v0c, o_mx0, o_mn0),
        (1, KS[1], i1, t1, v1c, o_mx1, o_mn1),
        (2, KS[2], i2, t2, v2c, o_mx2, o_mn2),
    ]
    for si, k, iflat, tab, vc, o_mx, o_mn in scales:
        gpts = GROUP_ROWS // k
        ngroups = WPTS // gpts
        pltpu.sync_copy(vc.at[pl.ds(base_pt, WPTS)], v_v)

        def load_grp(g, idx_v, sem, rows_v, iflat=iflat, tab=tab, k=k):
            pltpu.sync_copy(
                iflat.at[pl.ds((base_pt + g * gpts) * k, GROUP_ROWS)], idx_v)
            pltpu.async_copy(tab.at[idx_v], rows_v, sem)

        def reduce_grp(g, rows_v, accs, k=k, gpts=gpts):
            for p in range(gpts):
                r0 = p * k
                pacc = []
                for c in range(4):
                    val = rows_v[r0, pl.ds(c * 16, 16)]
                    pacc += [val, val, val, val * val]

                def red(j, a, r0=r0):
                    out = []
                    for c in range(4):
                        val = rows_v[r0 + j, pl.ds(c * 16, 16)]
                        m, mn, s1, s2 = a[4 * c:4 * c + 4]
                        out += [jnp.maximum(m, val), jnp.minimum(mn, val),
                                s1 + val, s2 + val * val]
                    return tuple(out)

                pacc = lax.fori_loop(1, k, red, tuple(pacc))
                lp = g * gpts + p
                nacc = []
                for c in range(4):
                    sl = pl.ds(c * 16, 16)
                    m, mn, s1, s2 = pacc[4 * c:4 * c + 4]
                    mx_v[lp, sl] = m
                    mn_v[lp, sl] = mn
                    vv = v_v[lp, sl]
                    a1, a2, ax = accs[3 * c], accs[3 * c + 1], accs[3 * c + 2]
                    nacc += [a1 + s1, a2 + s2, ax + vv * s1]
                accs = tuple(nacc)
            return accs

        load_grp(0, idx_a, sem_a, rows_a)
        zeros = tuple(jnp.zeros((16,), jnp.float32) for _ in range(12))

        def pair(g2, accs, k=k):
            g = 2 * g2
            load_grp(g + 1, idx_b, sem_b, rows_b)
            pltpu.make_async_copy(tab.at[idx_a], rows_a, sem_a).wait()
            accs = reduce_grp(g, rows_a, accs)

            @pl.when(g2 + 1 < ngroups // 2)
            def _():
                load_grp(g + 2, idx_a, sem_a, rows_a)

            pltpu.make_async_copy(tab.at[idx_b], rows_b, sem_b).wait()
            return reduce_grp(g + 1, rows_b, accs)

        accs = lax.fori_loop(0, ngroups // 2, pair, zeros)

        rows = pl.ds(base_pt, WPTS)
        pltpu.sync_copy(mx_v, o_mx.at[rows])
        pltpu.sync_copy(mn_v, o_mn.at[rows])
        for c in range(4):
            sl = pl.ds(c * 16, 16)
            part_v[3 * si + 0, sl] = accs[3 * c]
            part_v[3 * si + 1, sl] = accs[3 * c + 1]
            part_v[3 * si + 2, sl] = accs[3 * c + 2]
    pltpu.sync_copy(part_v, o_part.at[wid])


def _stage2(i0, i1, i2, t0, t1, t2, v0c, v1c, v2c):
    mesh = plsc.VectorSubcoreMesh(core_axis_name="c", subcore_axis_name="s",
                                  num_cores=SC_CORES, num_subcores=SC_SUBCORES)
    fn = pl.kernel(
        _sc_body,
        out_type=[jax.ShapeDtypeStruct((CPTS, PER), jnp.float32)] * 6
        + [jax.ShapeDtypeStruct((NW, 9, PER), jnp.float32)],
        mesh=mesh,
        scratch_types=[
            pltpu.VMEM((GROUP_ROWS,), jnp.int32),
            pltpu.VMEM((GROUP_ROWS,), jnp.int32),
            pltpu.VMEM((GROUP_ROWS, PER), jnp.float32),
            pltpu.VMEM((GROUP_ROWS, PER), jnp.float32),
            pltpu.VMEM((WPTS, PER), jnp.float32),
            pltpu.VMEM((WPTS, PER), jnp.float32),
            pltpu.VMEM((WPTS, PER), jnp.float32),
            pltpu.VMEM((9, PER), jnp.float32),
            pltpu.SemaphoreType.DMA,
            pltpu.SemaphoreType.DMA,
        ],
        compiler_params=pltpu.CompilerParams(use_tc_tiling_on_sc=False),
    )
    return fn(i0, i1, i2, t0, t1, t2, v0c, v1c, v2c)


_RBLK = 1024
_NBLK = BN // _RBLK


def _rows_spec(width):
    return pl.BlockSpec((_RBLK, width), lambda i: (i, 0))


def _fixed_spec(shape):
    nd = len(shape)
    return pl.BlockSpec(shape, lambda i: (0,) * nd)


def _s3b_body(part, uvst, gam, bet, mx0, mn0, v0, mx1, mn1, v1, mx2, mn2, v2,
              h_ref, g_ref, hs_ref):
    psum = jnp.sum(part[...], axis=0, keepdims=True)
    parts = []
    groups = ((mx0, mn0, v0), (mx1, mn1, v1), (mx2, mn2, v2))
    for i, (mx, mn, v) in enumerate(groups):
        k = KS[i]
        cnt = jnp.float32(BN * k)
        S1 = psum[:, (3 * i + 0) * PER:(3 * i + 1) * PER]
        S2 = psum[:, (3 * i + 1) * PER:(3 * i + 2) * PER]
        X = psum[:, (3 * i + 2) * PER:(3 * i + 3) * PER]
        V1 = uvst[2 * i:2 * i + 1, :]
        V2 = uvst[2 * i + 1:2 * i + 2, :]
        mean = (S1 + k * V1) / cnt
        e2 = (S2 + 2.0 * X + k * V2) / cnt
        var = e2 - mean * mean
        gamma = gam[i:i + 1, :]
        beta = bet[i:i + 1, :]
        pre = jnp.where(gamma >= 0.0, mx[...], mn[...]) + v[...]
        y = (pre - mean) * (gamma / jnp.sqrt(var + 1e-5)) + beta
        parts.append(jnp.where(y > 0.0, y, 0.2 * y))
    h = jnp.concatenate(parts, axis=1)
    h_ref[...] = h

    @pl.when(pl.program_id(0) == 0)
    def _():
        g_ref[...] = jnp.zeros_like(g_ref)
        hs_ref[...] = jnp.zeros_like(hs_ref)

    g_ref[...] += lax.dot_general(h, h, (((0,), (0,)), ((), ())),
                                  preferred_element_type=jnp.float32,
                                  precision=_HIGH)
    hs_ref[...] += jnp.sum(h, axis=0, keepdims=True)


def _s3cd_body(h_ref, g_ref, hs_ref, wf_ref, bf_ref, gf_ref, betaf_ref,
               out_ref):
    wf = wf_ref[...]
    bf = bf_ref[...]
    hs = hs_ref[...]
    zs = lax.dot_general(hs, wf, (((1,), (1,)), ((), ())),
                         preferred_element_type=jnp.float32,
                         precision=_HIGH) + jnp.float32(BN) * bf
    m_wg = lax.dot_general(wf, g_ref[...], (((1,), (0,)), ((), ())),
                           preferred_element_type=jnp.float32,
                           precision=_HIGH)
    diag_col = jnp.sum(m_wg * wf, axis=1, keepdims=True)
    eye = (lax.broadcasted_iota(jnp.int32, (OUT_C, OUT_C), 0)
           == lax.broadcasted_iota(jnp.int32, (OUT_C, OUT_C), 1)
           ).astype(jnp.float32)
    diag = lax.dot_general(diag_col, eye, (((0,), (0,)), ((), ())),
                           preferred_element_type=jnp.float32,
                           precision=_HIGH)
    wh = lax.dot_general(hs, wf, (((1,), (1,)), ((), ())),
                         preferred_element_type=jnp.float32,
                         precision=_HIGH)
    z2s = diag + 2.0 * bf * wh + jnp.float32(BN) * bf * bf
    mean = zs / jnp.float32(BN)
    var = z2s / jnp.float32(BN) - mean * mean

    z = lax.dot_general(h_ref[...], wf, (((1,), (1,)), ((), ())),
                        preferred_element_type=jnp.float32,
                        precision=_HIGH) + bf
    z = (z - mean) * (gf_ref[...] / jnp.sqrt(var + 1e-5)) + betaf_ref[...]
    z = 0.5 * z * (1.0 + lax.erf(z * jnp.float32(0.7071067811865476)))
    eye_n = (lax.broadcasted_iota(jnp.int32, (_RBLK, _RBLK), 0)
             == lax.broadcasted_iota(jnp.int32, (_RBLK, _RBLK), 1)
             ).astype(jnp.float32)
    out_ref[0] = lax.dot_general(z, eye_n, (((0,), (0,)), ((), ())),
                                 preferred_element_type=jnp.float32,
                                 precision=_HIGH)


def _stage3(mx, mn, part, uvst, v_rows, gam, bet, wf, bf, gf, betaf):
    h, gmat, hs = pl.pallas_call(
        _s3b_body,
        grid=(_NBLK,),
        in_specs=[_fixed_spec((CHUNKS * NW, 9 * PER)), _fixed_spec((6, PER)),
                  _fixed_spec((3, PER)), _fixed_spec((3, PER))]
        + [_rows_spec(PER)] * 9,
        out_specs=[_rows_spec(OUT_C), _fixed_spec((OUT_C, OUT_C)),
                   _fixed_spec((1, OUT_C))],
        out_shape=[jax.ShapeDtypeStruct((BN, OUT_C), jnp.float32),
                   jax.ShapeDtypeStruct((OUT_C, OUT_C), jnp.float32),
                   jax.ShapeDtypeStruct((1, OUT_C), jnp.float32)],
    )(part, uvst, gam, bet, mx[0], mn[0], v_rows[0],
      mx[1], mn[1], v_rows[1], mx[2], mn[2], v_rows[2])

    return pl.pallas_call(
        _s3cd_body,
        grid=(_NBLK,),
        in_specs=[_rows_spec(OUT_C), _fixed_spec((OUT_C, OUT_C)),
                  _fixed_spec((1, OUT_C)), _fixed_spec((OUT_C, OUT_C)),
                  _fixed_spec((1, OUT_C)), _fixed_spec((1, OUT_C)),
                  _fixed_spec((1, OUT_C))],
        out_specs=pl.BlockSpec((1, OUT_C, _RBLK), lambda i: (i, 0, 0)),
        out_shape=jax.ShapeDtypeStruct((B, OUT_C, N), jnp.float32),
    )(h, gmat, hs, wf, bf, gf, betaf)


def kernel(x, W0, gamma0, beta0, W1, gamma1, beta1, W2, gamma2, beta2,
           Wf, bf, gf, betaf):
    xt = jnp.transpose(x, (0, 2, 1))
    w_all = jnp.stack([W0, W1, W2])
    u, v, uvst = _uv(xt, w_all)
    tabs = [u[:, i].reshape(BN, PER) for i in range(3)]
    v_rows = [v[:, i].reshape(BN, PER) for i in range(3)]

    mx_c = [[] for _ in range(3)]
    mn_c = [[] for _ in range(3)]
    parts = []
    for c in range(CHUNKS):
        idxg = _topk(x[c * CB:(c + 1) * CB], xt[c * CB:(c + 1) * CB], c)
        i2 = idxg.reshape(-1)
        i1 = idxg[:, :, :KS[1]].reshape(-1)
        i0 = idxg[:, :, :KS[0]].reshape(-1)
        vc = [vr[c * CPTS:(c + 1) * CPTS] for vr in v_rows]
        outs = _stage2(i0, i1, i2, tabs[0], tabs[1], tabs[2],
                       vc[0], vc[1], vc[2])
        for si in range(3):
            mx_c[si].append(outs[2 * si])
            mn_c[si].append(outs[2 * si + 1])
        parts.append(outs[6])

    mx = [jnp.concatenate(mx_c[si], axis=0) for si in range(3)]
    mn = [jnp.concatenate(mn_c[si], axis=0) for si in range(3)]
    part = jnp.concatenate(parts, axis=0).reshape(CHUNKS * NW, 9 * PER)

    gam = jnp.stack([gamma0, gamma1, gamma2])
    bet = jnp.stack([beta0, beta1, beta2])
    return _stage3(mx, mn, part, uvst, v_rows, gam, bet, Wf,
                   bf.reshape(1, OUT_C), gf.reshape(1, OUT_C),
                   betaf.reshape(1, OUT_C))


# speedup vs baseline: 1.2162x; 1.0369x over previous
import jax
import jax.numpy as jnp
from jax import lax
from jax.experimental import pallas as pl
from jax.experimental.pallas import tpu as pltpu
from jax.experimental.pallas import tpu_sc as plsc

B = 8
C = 128
N = 1024
KS = (8, 16, 32)
PER = 64
OUT_C = 192
BN = B * N
CHUNKS = 4
CB = B // CHUNKS
CPTS = CB * N

SC_CORES = 2
SC_SUBCORES = 16
NW = SC_CORES * SC_SUBCORES
WPTS = CPTS // NW
GROUP_ROWS = 128

_HIGH = jax.lax.Precision.HIGHEST


def _uv_body(xt_ref, w_ref, u_ref, v_ref, st_ref):
    xt = xt_ref[0]

    @pl.when(pl.program_id(0) == 0)
    def _():
        st_ref[...] = jnp.zeros_like(st_ref)

    acc = []
    for i in range(3):
        w = w_ref[i]
        a = w[:, :C]
        bm = w[:, C:] - a
        u_ref[0, i] = lax.dot_general(xt, a, (((1,), (1,)), ((), ())),
                                      preferred_element_type=jnp.float32,
                                      precision=_HIGH)
        vv = lax.dot_general(xt, bm, (((1,), (1,)), ((), ())),
                             preferred_element_type=jnp.float32,
                             precision=_HIGH)
        v_ref[0, i] = vv
        acc.append(jnp.sum(vv, axis=0, keepdims=True))
        acc.append(jnp.sum(vv * vv, axis=0, keepdims=True))
    st_ref[...] += jnp.concatenate(acc, axis=0)


def _uv(xt, w_all):
    return pl.pallas_call(
        _uv_body,
        grid=(B,),
        in_specs=[
            pl.BlockSpec((1, N, C), lambda b: (b, 0, 0)),
            pl.BlockSpec((3, PER, 2 * C), lambda b: (0, 0, 0)),
        ],
        out_specs=[
            pl.BlockSpec((1, 3, N, PER), lambda b: (b, 0, 0, 0)),
            pl.BlockSpec((1, 3, N, PER), lambda b: (b, 0, 0, 0)),
            pl.BlockSpec((6, PER), lambda b: (0, 0)),
        ],
        out_shape=[
            jax.ShapeDtypeStruct((B, 3, N, PER), jnp.float32),
            jax.ShapeDtypeStruct((B, 3, N, PER), jnp.float32),
            jax.ShapeDtypeStruct((6, PER), jnp.float32),
        ],
    )(xt, w_all)


def _make_top_body(chunk):
    def body(x_ref, xt_ref, idx_ref, d_ref):
        for s in range(CB):
            x = x_ref[s]
            xt = xt_ref[s]
            ah = xt.astype(jnp.bfloat16)
            bh = x.astype(jnp.bfloat16)
            inner = -2.0 * lax.dot_general(ah, bh, (((1,), (0,)), ((), ())),
                                           preferred_element_type=jnp.float32)
            xx_row = jnp.sum(x * x, axis=0, keepdims=True)
            xx_col = jnp.sum(xt * xt, axis=1, keepdims=True)
            d_ref[pl.ds(s * N, N), :] = (-xx_row - inner) - xx_col

        cols = lax.broadcasted_iota(jnp.int32, (CB * N, N), 1)
        jj = lax.broadcasted_iota(jnp.int32, (CB * N, KS[-1]), 1)
        rowb = (chunk * CB * N
                + N * (lax.broadcasted_iota(jnp.int32, (CB * N, 1), 0) // N))

        def step(j, idxc):
            d = d_ref[...]
            rm = jnp.max(d, axis=1, keepdims=True)
            am = jnp.min(jnp.where(d == rm, cols, N), axis=1, keepdims=True)
            d_ref[...] = jnp.where(cols == am, jnp.float32(-1e30), d)
            return jnp.where(jj == j, am + rowb, idxc)

        idxc = lax.fori_loop(0, KS[-1], step,
                             jnp.zeros((CB * N, KS[-1]), jnp.int32))
        idx_ref[...] = idxc.reshape(CB, N, KS[-1])
    return body


def _topk(x, xt, chunk):
    return pl.pallas_call(
        _make_top_body(chunk),
        grid=(1,),
        in_specs=[
            pl.BlockSpec((CB, C, N), lambda i: (chunk, 0, 0)),
            pl.BlockSpec((CB, N, C), lambda i: (chunk, 0, 0)),
        ],
        out_specs=pl.BlockSpec((CB, N, KS[-1]), lambda i: (0, 0, 0)),
        out_shape=jax.ShapeDtypeStruct((CB, N, KS[-1]), jnp.int32),
        scratch_shapes=[pltpu.VMEM((CB * N, N), jnp.float32)],
    )(x, xt)


def _make_sc_body(chunk):
  def _sc_body(i0, i1, i2, t0, t1, t2, v0c, v1c, v2c,
             o_mx0, o_mn0, o_mx1, o_mn1, o_mx2, o_mn2, o_part,
             idx_all, rows_a, rows_b, v_v, mx_v, mn_v, part_v,
             sem_a, sem_b):
    wid = lax.axis_index("s") * SC_CORES + lax.axis_index("c")
    base_pt = wid * WPTS
    voff = chunk * CPTS + base_pt

    scales = [
        (0, KS[0], i0, t0, v0c, o_mx0, o_mn0),
        (1, KS[1], i1, t1, v1c, o_mx1, o_mn1),
        (2, KS[2], i2, t2, v2c, o_mx2, o_mn2),
    ]
    for si, k, iflat, tab, vc, o_mx, o_mn in scales:
        gpts = GROUP_ROWS // k
        ngroups = WPTS // gpts
        pltpu.sync_copy(vc.at[pl.ds(voff, WPTS)], v_v)
        pltpu.sync_copy(iflat.at[pl.ds(base_pt * k, WPTS * k)],
                        idx_all.at[pl.ds(0, WPTS * k)])

        def load_grp(g, sem, rows_v, tab=tab):
            isl = idx_all.at[pl.ds(g * GROUP_ROWS, GROUP_ROWS)]
            pltpu.async_copy(tab.at[isl], rows_v, sem)

        def wait_grp(g, sem, rows_v, tab=tab):
            isl = idx_all.at[pl.ds(g * GROUP_ROWS, GROUP_ROWS)]
            pltpu.make_async_copy(tab.at[isl], rows_v, sem).wait()

        def reduce_grp(g, rows_v, accs, k=k, gpts=gpts):
            for p in range(gpts):
                r0 = p * k
                neg = jnp.full((16,), -3.4e38, jnp.float32)
                pos = jnp.full((16,), 3.4e38, jnp.float32)
                zero = jnp.zeros((16,), jnp.float32)
                pacc = [neg, pos, zero, zero] * 4

                def red(j4, a, r0=r0):
                    out = list(a)
                    for u in range(4):
                        r = r0 + j4 * 4 + u
                        for c in range(4):
                            val = rows_v[r, pl.ds(c * 16, 16)]
                            m, mn, s1, s2 = out[4 * c:4 * c + 4]
                            out[4 * c:4 * c + 4] = [
                                jnp.maximum(m, val), jnp.minimum(mn, val),
                                s1 + val, s2 + val * val]
                    return tuple(out)

                pacc = lax.fori_loop(0, k // 4, red, tuple(pacc))
                lp = g * gpts + p
                nacc = []
                for c in range(4):
                    sl = pl.ds(c * 16, 16)
                    m, mn, s1, s2 = pacc[4 * c:4 * c + 4]
                    mx_v[lp, sl] = m
                    mn_v[lp, sl] = mn
                    vv = v_v[lp, sl]
                    a1, a2, ax = accs[3 * c], accs[3 * c + 1], accs[3 * c + 2]
                    nacc += [a1 + s1, a2 + s2, ax + vv * s1]
                accs = tuple(nacc)
            return accs

        load_grp(0, sem_a, rows_a)
        zeros = tuple(jnp.zeros((16,), jnp.float32) for _ in range(12))

        def pair(g2, accs, k=k):
            g = 2 * g2
            load_grp(g + 1, sem_b, rows_b)
            wait_grp(g, sem_a, rows_a)
            accs = reduce_grp(g, rows_a, accs)

            @pl.when(g2 + 1 < ngroups // 2)
            def _():
                load_grp(g + 2, sem_a, rows_a)

            wait_grp(g + 1, sem_b, rows_b)
            return reduce_grp(g + 1, rows_b, accs)

        accs = lax.fori_loop(0, ngroups // 2, pair, zeros)

        rows = pl.ds(base_pt, WPTS)
        pltpu.sync_copy(mx_v, o_mx.at[rows])
        pltpu.sync_copy(mn_v, o_mn.at[rows])
        for c in range(4):
            sl = pl.ds(c * 16, 16)
            part_v[3 * si + 0, sl] = accs[3 * c]
            part_v[3 * si + 1, sl] = accs[3 * c + 1]
            part_v[3 * si + 2, sl] = accs[3 * c + 2]
    pltpu.sync_copy(part_v, o_part.at[wid])
  return _sc_body


def _stage2(chunk, i0, i1, i2, t0, t1, t2, v0c, v1c, v2c):
    mesh = plsc.VectorSubcoreMesh(core_axis_name="c", subcore_axis_name="s",
                                  num_cores=SC_CORES, num_subcores=SC_SUBCORES)
    fn = pl.kernel(
        _make_sc_body(chunk),
        out_type=[jax.ShapeDtypeStruct((CPTS, PER), jnp.float32)] * 6
        + [jax.ShapeDtypeStruct((NW, 9, PER), jnp.float32)],
        mesh=mesh,
        scratch_types=[
            pltpu.VMEM((WPTS * KS[-1],), jnp.int32),
            pltpu.VMEM((GROUP_ROWS, PER), jnp.float32),
            pltpu.VMEM((GROUP_ROWS, PER), jnp.float32),
            pltpu.VMEM((WPTS, PER), jnp.float32),
            pltpu.VMEM((WPTS, PER), jnp.float32),
            pltpu.VMEM((WPTS, PER), jnp.float32),
            pltpu.VMEM((9, PER), jnp.float32),
            pltpu.SemaphoreType.DMA,
            pltpu.SemaphoreType.DMA,
        ],
        compiler_params=pltpu.CompilerParams(use_tc_tiling_on_sc=False),
    )
    return fn(i0, i1, i2, t0, t1, t2, v0c, v1c, v2c)


_RBLK = 1024
_NBLK = BN // _RBLK


def _rows_spec(width):
    return pl.BlockSpec((_RBLK, width), lambda i: (i, 0))


def _fixed_spec(shape):
    nd = len(shape)
    return pl.BlockSpec(shape, lambda i: (0,) * nd)


def _s3b_body(part, uvst, gam, bet, mx0, mn0, v0, mx1, mn1, v1, mx2, mn2, v2,
              h_ref, g_ref, hs_ref):
    psum = jnp.sum(part[...], axis=0, keepdims=True)
    parts = []
    groups = ((mx0, mn0, v0), (mx1, mn1, v1), (mx2, mn2, v2))
    for i, (mx, mn, v) in enumerate(groups):
        k = KS[i]
        cnt = jnp.float32(BN * k)
        S1 = psum[:, (3 * i + 0) * PER:(3 * i + 1) * PER]
        S2 = psum[:, (3 * i + 1) * PER:(3 * i + 2) * PER]
        X = psum[:, (3 * i + 2) * PER:(3 * i + 3) * PER]
        V1 = uvst[2 * i:2 * i + 1, :]
        V2 = uvst[2 * i + 1:2 * i + 2, :]
        mean = (S1 + k * V1) / cnt
        e2 = (S2 + 2.0 * X + k * V2) / cnt
        var = e2 - mean * mean
        gamma = gam[i:i + 1, :]
        beta = bet[i:i + 1, :]
        pre = jnp.where(gamma >= 0.0, mx[...], mn[...]) + v[...]
        y = (pre - mean) * (gamma / jnp.sqrt(var + 1e-5)) + beta
        parts.append(jnp.where(y > 0.0, y, 0.2 * y))
    h = jnp.concatenate(parts, axis=1)
    h_ref[...] = h

    @pl.when(pl.program_id(0) == 0)
    def _():
        g_ref[...] = jnp.zeros_like(g_ref)
        hs_ref[...] = jnp.zeros_like(hs_ref)

    g_ref[...] += lax.dot_general(h, h, (((0,), (0,)), ((), ())),
                                  preferred_element_type=jnp.float32,
                                  precision=_HIGH)
    hs_ref[...] += jnp.sum(h, axis=0, keepdims=True)


def _s3cd_body(h_ref, g_ref, hs_ref, wf_ref, bf_ref, gf_ref, betaf_ref,
               out_ref):
    wf = wf_ref[...]
    bf = bf_ref[...]
    hs = hs_ref[...]
    zs = lax.dot_general(hs, wf, (((1,), (1,)), ((), ())),
                         preferred_element_type=jnp.float32,
                         precision=_HIGH) + jnp.float32(BN) * bf
    m_wg = lax.dot_general(wf, g_ref[...], (((1,), (0,)), ((), ())),
                           preferred_element_type=jnp.float32,
                           precision=_HIGH)
    diag_col = jnp.sum(m_wg * wf, axis=1, keepdims=True)
    eye = (lax.broadcasted_iota(jnp.int32, (OUT_C, OUT_C), 0)
           == lax.broadcasted_iota(jnp.int32, (OUT_C, OUT_C), 1)
           ).astype(jnp.float32)
    diag = lax.dot_general(diag_col, eye, (((0,), (0,)), ((), ())),
                           preferred_element_type=jnp.float32,
                           precision=_HIGH)
    wh = lax.dot_general(hs, wf, (((1,), (1,)), ((), ())),
                         preferred_element_type=jnp.float32,
                         precision=_HIGH)
    z2s = diag + 2.0 * bf * wh + jnp.float32(BN) * bf * bf
    mean = zs / jnp.float32(BN)
    var = z2s / jnp.float32(BN) - mean * mean

    z = lax.dot_general(h_ref[...], wf, (((1,), (1,)), ((), ())),
                        preferred_element_type=jnp.float32,
                        precision=_HIGH) + bf
    z = (z - mean) * (gf_ref[...] / jnp.sqrt(var + 1e-5)) + betaf_ref[...]
    z = 0.5 * z * (1.0 + lax.erf(z * jnp.float32(0.7071067811865476)))
    eye_n = (lax.broadcasted_iota(jnp.int32, (_RBLK, _RBLK), 0)
             == lax.broadcasted_iota(jnp.int32, (_RBLK, _RBLK), 1)
             ).astype(jnp.float32)
    out_ref[0] = lax.dot_general(z, eye_n, (((0,), (0,)), ((), ())),
                                 preferred_element_type=jnp.float32,
                                 precision=_HIGH)


def _stage3(mx, mn, part, uvst, v_rows, gam, bet, wf, bf, gf, betaf):
    h, gmat, hs = pl.pallas_call(
        _s3b_body,
        grid=(_NBLK,),
        in_specs=[_fixed_spec((CHUNKS * NW, 9 * PER)), _fixed_spec((6, PER)),
                  _fixed_spec((3, PER)), _fixed_spec((3, PER))]
        + [_rows_spec(PER)] * 9,
        out_specs=[_rows_spec(OUT_C), _fixed_spec((OUT_C, OUT_C)),
                   _fixed_spec((1, OUT_C))],
        out_shape=[jax.ShapeDtypeStruct((BN, OUT_C), jnp.float32),
                   jax.ShapeDtypeStruct((OUT_C, OUT_C), jnp.float32),
                   jax.ShapeDtypeStruct((1, OUT_C), jnp.float32)],
    )(part, uvst, gam, bet, mx[0], mn[0], v_rows[0],
      mx[1], mn[1], v_rows[1], mx[2], mn[2], v_rows[2])

    return pl.pallas_call(
        _s3cd_body,
        grid=(_NBLK,),
        in_specs=[_rows_spec(OUT_C), _fixed_spec((OUT_C, OUT_C)),
                  _fixed_spec((1, OUT_C)), _fixed_spec((OUT_C, OUT_C)),
                  _fixed_spec((1, OUT_C)), _fixed_spec((1, OUT_C)),
                  _fixed_spec((1, OUT_C))],
        out_specs=pl.BlockSpec((1, OUT_C, _RBLK), lambda i: (i, 0, 0)),
        out_shape=jax.ShapeDtypeStruct((B, OUT_C, N), jnp.float32),
    )(h, gmat, hs, wf, bf, gf, betaf)


def kernel(x, W0, gamma0, beta0, W1, gamma1, beta1, W2, gamma2, beta2,
           Wf, bf, gf, betaf):
    xt = jnp.transpose(x, (0, 2, 1))
    w_all = jnp.stack([W0, W1, W2])
    u, v, uvst = _uv(xt, w_all)
    tabs = [u[:, i].reshape(BN, PER) for i in range(3)]
    v_rows = [v[:, i].reshape(BN, PER) for i in range(3)]

    mx_c = [[] for _ in range(3)]
    mn_c = [[] for _ in range(3)]
    parts = []
    for c in range(CHUNKS):
        idxg = _topk(x, xt, c)
        i2 = idxg.reshape(-1)
        i1 = idxg[:, :, :KS[1]].reshape(-1)
        i0 = idxg[:, :, :KS[0]].reshape(-1)
        outs = _stage2(c, i0, i1, i2, tabs[0], tabs[1], tabs[2],
                       v_rows[0], v_rows[1], v_rows[2])
        for si in range(3):
            mx_c[si].append(outs[2 * si])
            mn_c[si].append(outs[2 * si + 1])
        parts.append(outs[6])

    mx = [jnp.concatenate(mx_c[si], axis=0) for si in range(3)]
    mn = [jnp.concatenate(mn_c[si], axis=0) for si in range(3)]
    part = jnp.concatenate(parts, axis=0).reshape(CHUNKS * NW, 9 * PER)

    gam = jnp.stack([gamma0, gamma1, gamma2])
    bet = jnp.stack([beta0, beta1, beta2])
    return _stage3(mx, mn, part, uvst, v_rows, gam, bet, Wf,
                   bf.reshape(1, OUT_C), gf.reshape(1, OUT_C),
                   betaf.reshape(1, OUT_C))


# speedup vs baseline: 1.2678x; 1.0424x over previous
import jax
import jax.numpy as jnp
from jax import lax
from jax.experimental import pallas as pl
from jax.experimental.pallas import tpu as pltpu
from jax.experimental.pallas import tpu_sc as plsc

B = 8
C = 128
N = 1024
KS = (8, 16, 32)
PER = 64
OUT_C = 192
BN = B * N
CHUNKS = 4
CB = B // CHUNKS
CPTS = CB * N

SC_CORES = 2
SC_SUBCORES = 16
NW = SC_CORES * SC_SUBCORES
WPTS = CPTS // NW
GROUP_ROWS = 128

_HIGH = jax.lax.Precision.HIGHEST


def _uv_body(xt_ref, w_ref, u_ref, v_ref, st_ref):
    xt = xt_ref[0]

    @pl.when(pl.program_id(0) == 0)
    def _():
        st_ref[...] = jnp.zeros_like(st_ref)

    acc = []
    for i in range(3):
        w = w_ref[i]
        a = w[:, :C]
        bm = w[:, C:] - a
        u_ref[0, i] = lax.dot_general(xt, a, (((1,), (1,)), ((), ())),
                                      preferred_element_type=jnp.float32,
                                      precision=_HIGH)
        vv = lax.dot_general(xt, bm, (((1,), (1,)), ((), ())),
                             preferred_element_type=jnp.float32,
                             precision=_HIGH)
        v_ref[0, i] = vv
        acc.append(jnp.sum(vv, axis=0, keepdims=True))
        acc.append(jnp.sum(vv * vv, axis=0, keepdims=True))
    st_ref[...] += jnp.concatenate(acc, axis=0)


def _uv(xt, w_all):
    return pl.pallas_call(
        _uv_body,
        grid=(B,),
        in_specs=[
            pl.BlockSpec((1, N, C), lambda b: (b, 0, 0)),
            pl.BlockSpec((3, PER, 2 * C), lambda b: (0, 0, 0)),
        ],
        out_specs=[
            pl.BlockSpec((1, 3, N, PER), lambda b: (b, 0, 0, 0)),
            pl.BlockSpec((1, 3, N, PER), lambda b: (b, 0, 0, 0)),
            pl.BlockSpec((6, PER), lambda b: (0, 0)),
        ],
        out_shape=[
            jax.ShapeDtypeStruct((B, 3, N, PER), jnp.float32),
            jax.ShapeDtypeStruct((B, 3, N, PER), jnp.float32),
            jax.ShapeDtypeStruct((6, PER), jnp.float32),
        ],
    )(xt, w_all)


def _make_top_body(chunk):
    def body(x_ref, xt_ref, idx_ref, d_ref):
        for s in range(CB):
            x = x_ref[s]
            xt = xt_ref[s]
            ah = xt.astype(jnp.bfloat16)
            bh = x.astype(jnp.bfloat16)
            inner = -2.0 * lax.dot_general(ah, bh, (((1,), (0,)), ((), ())),
                                           preferred_element_type=jnp.float32)
            xx_row = jnp.sum(x * x, axis=0, keepdims=True)
            xx_col = jnp.sum(xt * xt, axis=1, keepdims=True)
            d_ref[pl.ds(s * N, N), :] = (-xx_row - inner) - xx_col

        cols = lax.broadcasted_iota(jnp.int32, (CB * N, N), 1)
        jj = lax.broadcasted_iota(jnp.int32, (CB * N, KS[-1]), 1)
        rowb = (chunk * CB * N
                + N * (lax.broadcasted_iota(jnp.int32, (CB * N, 1), 0) // N))

        def step(j, idxc):
            d = d_ref[...]
            rm = jnp.max(d, axis=1, keepdims=True)
            am = jnp.min(jnp.where(d == rm, cols, N), axis=1, keepdims=True)
            d_ref[...] = jnp.where(cols == am, jnp.float32(-1e30), d)
            return jnp.where(jj == j, am + rowb, idxc)

        idxc = lax.fori_loop(0, KS[-1], step,
                             jnp.zeros((CB * N, KS[-1]), jnp.int32))
        idx_ref[...] = idxc.reshape(CB, N, KS[-1])
    return body


def _topk(x, xt, chunk):
    return pl.pallas_call(
        _make_top_body(chunk),
        grid=(1,),
        in_specs=[
            pl.BlockSpec((CB, C, N), lambda i: (chunk, 0, 0)),
            pl.BlockSpec((CB, N, C), lambda i: (chunk, 0, 0)),
        ],
        out_specs=pl.BlockSpec((CB, N, KS[-1]), lambda i: (0, 0, 0)),
        out_shape=jax.ShapeDtypeStruct((CB, N, KS[-1]), jnp.int32),
        scratch_shapes=[pltpu.VMEM((CB * N, N), jnp.float32)],
    )(x, xt)


def _make_sc_body(chunk):
  def _sc_body(i0, i1, i2, t0, t1, t2, v0c, v1c, v2c, gam_all,
             o_e0, o_e1, o_e2, o_part,
             idx_all, rows_a, rows_b, v_v, gam_v, ex_v, part_v,
             sem_a, sem_b):
    wid = lax.axis_index("s") * SC_CORES + lax.axis_index("c")
    base_pt = wid * WPTS
    voff = chunk * CPTS + base_pt

    scales = [
        (0, KS[0], i0, t0, v0c, o_e0),
        (1, KS[1], i1, t1, v1c, o_e1),
        (2, KS[2], i2, t2, v2c, o_e2),
    ]
    for si, k, iflat, tab, vc, o_ex in scales:
        gpts = GROUP_ROWS // k
        ngroups = WPTS // gpts
        pltpu.sync_copy(vc.at[pl.ds(voff, WPTS)], v_v)
        pltpu.sync_copy(gam_all.at[si], gam_v)
        pltpu.sync_copy(iflat.at[pl.ds(base_pt * k, WPTS * k)],
                        idx_all.at[pl.ds(0, WPTS * k)])
        gpos = [gam_v[pl.ds(c * 16, 16)] >= 0.0 for c in range(4)]

        def load_grp(g, sem, rows_v, tab=tab):
            isl = idx_all.at[pl.ds(g * GROUP_ROWS, GROUP_ROWS)]
            pltpu.async_copy(tab.at[isl], rows_v, sem)

        def wait_grp(g, sem, rows_v, tab=tab):
            isl = idx_all.at[pl.ds(g * GROUP_ROWS, GROUP_ROWS)]
            pltpu.make_async_copy(tab.at[isl], rows_v, sem).wait()

        def reduce_grp(g, rows_v, accs, k=k, gpts=gpts):
            for p in range(gpts):
                r0 = p * k
                neg = jnp.full((16,), -3.4e38, jnp.float32)
                pos = jnp.full((16,), 3.4e38, jnp.float32)
                zero = jnp.zeros((16,), jnp.float32)
                pacc = [neg, pos, zero, zero] * 4

                def red(j4, a, r0=r0):
                    out = list(a)
                    for u in range(4):
                        r = r0 + j4 * 4 + u
                        for c in range(4):
                            val = rows_v[r, pl.ds(c * 16, 16)]
                            m, mn, s1, s2 = out[4 * c:4 * c + 4]
                            out[4 * c:4 * c + 4] = [
                                jnp.maximum(m, val), jnp.minimum(mn, val),
                                s1 + val, s2 + val * val]
                    return tuple(out)

                pacc = lax.fori_loop(0, k // 4, red, tuple(pacc))
                lp = g * gpts + p
                nacc = []
                for c in range(4):
                    sl = pl.ds(c * 16, 16)
                    m, mn, s1, s2 = pacc[4 * c:4 * c + 4]
                    ex_v[lp, sl] = jnp.where(gpos[c], m, mn)
                    vv = v_v[lp, sl]
                    a1, a2, ax = accs[3 * c], accs[3 * c + 1], accs[3 * c + 2]
                    nacc += [a1 + s1, a2 + s2, ax + vv * s1]
                accs = tuple(nacc)
            return accs

        load_grp(0, sem_a, rows_a)
        zeros = tuple(jnp.zeros((16,), jnp.float32) for _ in range(12))

        def pair(g2, accs, k=k):
            g = 2 * g2
            load_grp(g + 1, sem_b, rows_b)
            wait_grp(g, sem_a, rows_a)
            accs = reduce_grp(g, rows_a, accs)

            @pl.when(g2 + 1 < ngroups // 2)
            def _():
                load_grp(g + 2, sem_a, rows_a)

            wait_grp(g + 1, sem_b, rows_b)
            return reduce_grp(g + 1, rows_b, accs)

        accs = lax.fori_loop(0, ngroups // 2, pair, zeros)

        rows = pl.ds(base_pt, WPTS)
        pltpu.sync_copy(ex_v, o_ex.at[rows])
        for c in range(4):
            sl = pl.ds(c * 16, 16)
            part_v[3 * si + 0, sl] = accs[3 * c]
            part_v[3 * si + 1, sl] = accs[3 * c + 1]
            part_v[3 * si + 2, sl] = accs[3 * c + 2]
    pltpu.sync_copy(part_v, o_part.at[wid])
  return _sc_body


def _stage2(chunk, i0, i1, i2, t0, t1, t2, v0c, v1c, v2c, gam):
    mesh = plsc.VectorSubcoreMesh(core_axis_name="c", subcore_axis_name="s",
                                  num_cores=SC_CORES, num_subcores=SC_SUBCORES)
    fn = pl.kernel(
        _make_sc_body(chunk),
        out_type=[jax.ShapeDtypeStruct((CPTS, PER), jnp.float32)] * 3
        + [jax.ShapeDtypeStruct((NW, 9, PER), jnp.float32)],
        mesh=mesh,
        scratch_types=[
            pltpu.VMEM((WPTS * KS[-1],), jnp.int32),
            pltpu.VMEM((GROUP_ROWS, PER), jnp.float32),
            pltpu.VMEM((GROUP_ROWS, PER), jnp.float32),
            pltpu.VMEM((WPTS, PER), jnp.float32),
            pltpu.VMEM((PER,), jnp.float32),
            pltpu.VMEM((WPTS, PER), jnp.float32),
            pltpu.VMEM((9, PER), jnp.float32),
            pltpu.SemaphoreType.DMA,
            pltpu.SemaphoreType.DMA,
        ],
        compiler_params=pltpu.CompilerParams(use_tc_tiling_on_sc=False),
    )
    return fn(i0, i1, i2, t0, t1, t2, v0c, v1c, v2c, gam)


_RBLK = 1024
_NBLK = BN // _RBLK


def _rows_spec(width):
    return pl.BlockSpec((_RBLK, width), lambda i: (i, 0))


def _fixed_spec(shape):
    nd = len(shape)
    return pl.BlockSpec(shape, lambda i: (0,) * nd)


def _s3b_body(part, uvst, gam, bet, e0, v0, e1, v1, e2, v2,
              h_ref, g_ref, hs_ref):
    psum = jnp.sum(part[...], axis=0, keepdims=True)
    parts = []
    groups = ((e0, v0), (e1, v1), (e2, v2))
    for i, (ex, v) in enumerate(groups):
        k = KS[i]
        cnt = jnp.float32(BN * k)
        S1 = psum[:, (3 * i + 0) * PER:(3 * i + 1) * PER]
        S2 = psum[:, (3 * i + 1) * PER:(3 * i + 2) * PER]
        X = psum[:, (3 * i + 2) * PER:(3 * i + 3) * PER]
        V1 = uvst[2 * i:2 * i + 1, :]
        V2 = uvst[2 * i + 1:2 * i + 2, :]
        mean = (S1 + k * V1) / cnt
        e2 = (S2 + 2.0 * X + k * V2) / cnt
        var = e2 - mean * mean
        gamma = gam[i:i + 1, :]
        beta = bet[i:i + 1, :]
        pre = ex[...] + v[...]
        y = (pre - mean) * (gamma / jnp.sqrt(var + 1e-5)) + beta
        parts.append(jnp.where(y > 0.0, y, 0.2 * y))
    h = jnp.concatenate(parts, axis=1)
    h_ref[...] = h

    @pl.when(pl.program_id(0) == 0)
    def _():
        g_ref[...] = jnp.zeros_like(g_ref)
        hs_ref[...] = jnp.zeros_like(hs_ref)

    g_ref[...] += lax.dot_general(h, h, (((0,), (0,)), ((), ())),
                                  preferred_element_type=jnp.float32,
                                  precision=_HIGH)
    hs_ref[...] += jnp.sum(h, axis=0, keepdims=True)


def _s3cd_body(h_ref, g_ref, hs_ref, wf_ref, bf_ref, gf_ref, betaf_ref,
               out_ref):
    wf = wf_ref[...]
    bf = bf_ref[...]
    hs = hs_ref[...]
    zs = lax.dot_general(hs, wf, (((1,), (1,)), ((), ())),
                         preferred_element_type=jnp.float32,
                         precision=_HIGH) + jnp.float32(BN) * bf
    m_wg = lax.dot_general(wf, g_ref[...], (((1,), (0,)), ((), ())),
                           preferred_element_type=jnp.float32,
                           precision=_HIGH)
    diag_col = jnp.sum(m_wg * wf, axis=1, keepdims=True)
    eye = (lax.broadcasted_iota(jnp.int32, (OUT_C, OUT_C), 0)
           == lax.broadcasted_iota(jnp.int32, (OUT_C, OUT_C), 1)
           ).astype(jnp.float32)
    diag = lax.dot_general(diag_col, eye, (((0,), (0,)), ((), ())),
                           preferred_element_type=jnp.float32,
                           precision=_HIGH)
    wh = lax.dot_general(hs, wf, (((1,), (1,)), ((), ())),
                         preferred_element_type=jnp.float32,
                         precision=_HIGH)
    z2s = diag + 2.0 * bf * wh + jnp.float32(BN) * bf * bf
    mean = zs / jnp.float32(BN)
    var = z2s / jnp.float32(BN) - mean * mean

    z = lax.dot_general(h_ref[...], wf, (((1,), (1,)), ((), ())),
                        preferred_element_type=jnp.float32,
                        precision=_HIGH) + bf
    z = (z - mean) * (gf_ref[...] / jnp.sqrt(var + 1e-5)) + betaf_ref[...]
    z = 0.5 * z * (1.0 + lax.erf(z * jnp.float32(0.7071067811865476)))
    eye_n = (lax.broadcasted_iota(jnp.int32, (_RBLK, _RBLK), 0)
             == lax.broadcasted_iota(jnp.int32, (_RBLK, _RBLK), 1)
             ).astype(jnp.float32)
    out_ref[0] = lax.dot_general(z, eye_n, (((0,), (0,)), ((), ())),
                                 preferred_element_type=jnp.float32,
                                 precision=_HIGH)


def _stage3(ex, part, uvst, v_rows, gam, bet, wf, bf, gf, betaf):
    h, gmat, hs = pl.pallas_call(
        _s3b_body,
        grid=(_NBLK,),
        in_specs=[_fixed_spec((CHUNKS * NW, 9 * PER)), _fixed_spec((6, PER)),
                  _fixed_spec((3, PER)), _fixed_spec((3, PER))]
        + [_rows_spec(PER)] * 6,
        out_specs=[_rows_spec(OUT_C), _fixed_spec((OUT_C, OUT_C)),
                   _fixed_spec((1, OUT_C))],
        out_shape=[jax.ShapeDtypeStruct((BN, OUT_C), jnp.float32),
                   jax.ShapeDtypeStruct((OUT_C, OUT_C), jnp.float32),
                   jax.ShapeDtypeStruct((1, OUT_C), jnp.float32)],
    )(part, uvst, gam, bet, ex[0], v_rows[0],
      ex[1], v_rows[1], ex[2], v_rows[2])

    return pl.pallas_call(
        _s3cd_body,
        grid=(_NBLK,),
        in_specs=[_rows_spec(OUT_C), _fixed_spec((OUT_C, OUT_C)),
                  _fixed_spec((1, OUT_C)), _fixed_spec((OUT_C, OUT_C)),
                  _fixed_spec((1, OUT_C)), _fixed_spec((1, OUT_C)),
                  _fixed_spec((1, OUT_C))],
        out_specs=pl.BlockSpec((1, OUT_C, _RBLK), lambda i: (i, 0, 0)),
        out_shape=jax.ShapeDtypeStruct((B, OUT_C, N), jnp.float32),
    )(h, gmat, hs, wf, bf, gf, betaf)


def kernel(x, W0, gamma0, beta0, W1, gamma1, beta1, W2, gamma2, beta2,
           Wf, bf, gf, betaf):
    xt = jnp.transpose(x, (0, 2, 1))
    w_all = jnp.stack([W0, W1, W2])
    u, v, uvst = _uv(xt, w_all)
    tabs = [u[:, i].reshape(BN, PER) for i in range(3)]
    v_rows = [v[:, i].reshape(BN, PER) for i in range(3)]

    gam = jnp.stack([gamma0, gamma1, gamma2])
    bet = jnp.stack([beta0, beta1, beta2])

    ex_c = [[] for _ in range(3)]
    parts = []
    for c in range(CHUNKS):
        idxg = _topk(x, xt, c)
        i2 = idxg.reshape(-1)
        i1 = idxg[:, :, :KS[1]].reshape(-1)
        i0 = idxg[:, :, :KS[0]].reshape(-1)
        outs = _stage2(c, i0, i1, i2, tabs[0], tabs[1], tabs[2],
                       v_rows[0], v_rows[1], v_rows[2], gam)
        for si in range(3):
            ex_c[si].append(outs[si])
        parts.append(outs[3])

    ex = [jnp.concatenate(ex_c[si], axis=0) for si in range(3)]
    part = jnp.concatenate(parts, axis=0).reshape(CHUNKS * NW, 9 * PER)

    return _stage3(ex, part, uvst, v_rows, gam, bet, Wf,
                   bf.reshape(1, OUT_C), gf.reshape(1, OUT_C),
                   betaf.reshape(1, OUT_C))
